# Initial kernel scaffold; baseline (speedup 1.0000x reference)
import jax
import jax.numpy as jnp
from jax.experimental import pallas as pl


def kernel(fushed_features, input_data, Ww, bw, Wa1, ba1, Wa2, ba2, adj_idx):
    raise NotImplementedError("write your pallas kernel here")


# speedup vs baseline: 2.6582x; 2.6582x over previous
import functools

import jax
import jax.numpy as jnp
from jax import lax
from jax.experimental import pallas as pl
from jax.experimental.pallas import tpu as pltpu
from jax.experimental.pallas import tpu_sc as plsc

B, N, K, C, SX, SO = 4, 400, 32, 6, 12, 12
F = 16
H = 48
CP = 8
R = 8
M = R * K
GRID = (B * N) // R
OUTW = C * F

NC, NS = 2, 16
NW = NC * NS
PER_W = (B * N * K) // NW
CH, CW = 16, 100


def _gather_body(table_hbm, idx_hbm, out_hbm, idx_v, rows_v, sem):
    wid = lax.axis_index("s") * NC + lax.axis_index("c")
    pltpu.sync_copy(idx_hbm.at[wid], idx_v)
    copies = [
        pltpu.async_copy(
            table_hbm.at[idx_v.at[j]],
            rows_v.at[pl.ds(j * CW, CW)],
            sem,
        )
        for j in range(CH)
    ]
    for cp in copies:
        cp.wait()
    pltpu.sync_copy(rows_v, out_hbm.at[pl.ds(wid * PER_W, PER_W)])


def _sc_gather(table, idx3):
    mesh = plsc.VectorSubcoreMesh(core_axis_name="c", subcore_axis_name="s")
    run = functools.partial(
        pl.kernel,
        out_type=jax.ShapeDtypeStruct((B * N * K, F), jnp.float32),
        mesh=mesh,
        scratch_types=[
            pltpu.VMEM((CH, CW), jnp.int32),
            pltpu.VMEM((PER_W, F), jnp.float32),
            pltpu.SemaphoreType.DMA,
        ],
        compiler_params=pltpu.CompilerParams(use_tc_tiling_on_sc=False),
    )(_gather_body)
    return run(table, idx3)


def _tc_body(x_ref, g_ref, wwt_ref, bw_ref, wa1xt_ref, wa1yt_ref, ba1_ref,
             wa2t_ref, ba2_ref, out_ref, acc_ref):
    i = pl.program_id(0)

    @pl.when(i == 0)
    def _init():
        acc_ref[...] = jnp.zeros_like(acc_ref)

    x = x_ref[...]
    g = g_ref[...]
    wwt = wwt_ref[...]
    bw = bw_ref[...]

    def leaky(v):
        return jnp.where(v >= 0, v, 0.5 * v)

    wh = leaky(jnp.dot(x, wwt, preferred_element_type=jnp.float32) + bw)
    whg = leaky(jnp.dot(g, wwt, preferred_element_type=jnp.float32) + bw)

    px = jnp.dot(wh, wa1xt_ref[...], preferred_element_type=jnp.float32)
    px = px + ba1_ref[...]
    py = jnp.dot(whg, wa1yt_ref[...], preferred_element_type=jnp.float32)

    row_of = lax.broadcasted_iota(jnp.int32, (M, R), 0) // K
    sel = (row_of == lax.broadcasted_iota(jnp.int32, (M, R), 1))
    px_rep = jnp.dot(sel.astype(jnp.float32), px,
                     preferred_element_type=jnp.float32)

    h = leaky(px_rep + py)
    att = leaky(jnp.dot(h, wa2t_ref[...], preferred_element_type=jnp.float32)
                + ba2_ref[...])

    lane = lax.broadcasted_iota(jnp.int32, (M, CP), 1)
    valid = lane < C
    att_m = jnp.where(valid, att, -1e30)
    mx = jnp.max(att_m, axis=1, keepdims=True)
    e = jnp.where(valid, jnp.exp(att - mx), 0.0)
    am = e / jnp.sum(e, axis=1, keepdims=True)

    p1 = (lax.broadcasted_iota(jnp.int32, (CP, OUTW), 1) // F
          == lax.broadcasted_iota(jnp.int32, (CP, OUTW), 0)).astype(jnp.float32)
    p2 = (lax.broadcasted_iota(jnp.int32, (F, OUTW), 1) % F
          == lax.broadcasted_iota(jnp.int32, (F, OUTW), 0)).astype(jnp.float32)
    ee = (jnp.dot(am, p1, preferred_element_type=jnp.float32)
          * jnp.dot(whg, p2, preferred_element_type=jnp.float32))
    out_ref[...] = jnp.sum(ee.reshape(R, K, OUTW), axis=1)

    nt = (((1,), (1,)), ((), ()))
    prob = lax.dot_general(am, am, nt, preferred_element_type=jnp.float32)
    gram = lax.dot_general(whg, whg, nt, preferred_element_type=jnp.float32)
    sq = jnp.sum(whg * whg, axis=1, keepdims=True)
    ones = jnp.ones((M, 1), dtype=jnp.float32)
    sq_cols = lax.dot_general(ones, sq, nt, preferred_element_type=jnp.float32)
    dist = -2.0 * gram + sq + sq_cols

    r1 = lax.broadcasted_iota(jnp.int32, (M, M), 0)
    r2 = lax.broadcasted_iota(jnp.int32, (M, M), 1)
    blk = (r1 // K) == (r2 // K)
    sign = jnp.where(dist <= 0.2, 1.0, -1.0)
    lp = jnp.log(jnp.clip(prob, 0.0001, 1.0 - 0.0001))
    loss_sum = jnp.sum(jnp.where(blk & (r1 != r2), -sign * lp, 0.0))
    dist_sum = jnp.sum(jnp.where(blk, dist, 0.0))
    wh_sum = jnp.sum(wh)

    acc_lane = lax.broadcasted_iota(jnp.int32, (1, 128), 1)
    vec = jnp.where(acc_lane == 0, loss_sum,
                    jnp.where(acc_lane == 1, dist_sum,
                              jnp.where(acc_lane == 2, wh_sum, 0.0)))
    acc_ref[...] += vec


_TC_GRID_SPEC = dict(
    grid=(GRID,),
    in_specs=[
        pl.BlockSpec((R, F), lambda i: (i, 0)),
        pl.BlockSpec((M, F), lambda i: (i, 0)),
        pl.BlockSpec((F, F), lambda i: (0, 0)),
        pl.BlockSpec((1, F), lambda i: (0, 0)),
        pl.BlockSpec((F, H), lambda i: (0, 0)),
        pl.BlockSpec((F, H), lambda i: (0, 0)),
        pl.BlockSpec((1, H), lambda i: (0, 0)),
        pl.BlockSpec((H, CP), lambda i: (0, 0)),
        pl.BlockSpec((1, CP), lambda i: (0, 0)),
    ],
    out_specs=[
        pl.BlockSpec((R, OUTW), lambda i: (i, 0)),
        pl.BlockSpec((1, 128), lambda i: (0, 0)),
    ],
    out_shape=[
        jax.ShapeDtypeStruct((B * N, OUTW), jnp.float32),
        jax.ShapeDtypeStruct((1, 128), jnp.float32),
    ],
)


def _tc_compute(xpad, g, wwt, bw, wa1xt, wa1yt, ba1, wa2t, ba2):
    return pl.pallas_call(_tc_body, **_TC_GRID_SPEC)(
        xpad, g, wwt, bw, wa1xt, wa1yt, ba1, wa2t, ba2)


def kernel(fushed_features, input_data, Ww, bw, Wa1, ba1, Wa2, ba2, adj_idx):
    xpad = jnp.pad(input_data.reshape(B * N, SX), ((0, 0), (0, F - SX)))
    base = (jnp.arange(B, dtype=jnp.int32) * N)[:, None, None]
    idx3 = (adj_idx.astype(jnp.int32) + base).reshape(NW, CH, CW)

    g = _sc_gather(xpad, idx3)

    wwt = jnp.zeros((F, F), jnp.float32).at[:SX, :SO].set(Ww.T)
    bw_p = jnp.pad(bw, (0, F - SO)).reshape(1, F)
    wa1xt = jnp.zeros((F, H), jnp.float32).at[:SO].set(Wa1[:, :SO].T)
    wa1yt = jnp.zeros((F, H), jnp.float32).at[:SO].set(Wa1[:, SO:].T)
    ba1_p = ba1.reshape(1, H)
    wa2t = jnp.zeros((H, CP), jnp.float32).at[:, :C].set(Wa2.T)
    ba2_p = jnp.pad(ba2, (0, CP - C)).reshape(1, CP)

    out_main, acc = _tc_compute(xpad, g, wwt, bw_p, wa1xt, wa1yt, ba1_p,
                                wa2t, ba2_p)

    output_data = out_main.reshape(B * N, C, F)[:, :, :SO].reshape(B, N, C, SO)
    cluster_loss = acc[0, 0] / (B * N)
    dist_mean = acc[0, 1] / (B * N * K * K)
    wh_mean = acc[0, 2] / (B * N * SO)
    return output_data, cluster_loss, dist_mean, wh_mean


# speedup vs baseline: 4.0464x; 1.5222x over previous
import functools

import jax
import jax.numpy as jnp
from jax import lax
from jax.experimental import pallas as pl
from jax.experimental.pallas import tpu as pltpu
from jax.experimental.pallas import tpu_sc as plsc

B, N, K, C, SX, SO = 4, 400, 32, 6, 12, 12
F = 16
H = 48
CP = 8
R = 4
M = R * K
RB = 4
NB = RB * R
MB = NB * K
GRID = (B * N) // NB
OUTW = C * F

NC, NS = 2, 16
NW = NC * NS
PER_W = (B * N * K) // NW
CH, CW = 16, 100


def _gather_body(table_hbm, idx_hbm, out_hbm, idx_v, rows_v, sem):
    wid = lax.axis_index("s") * NC + lax.axis_index("c")
    pltpu.sync_copy(idx_hbm.at[wid], idx_v)
    copies = [
        pltpu.async_copy(
            table_hbm.at[idx_v.at[j]],
            rows_v.at[pl.ds(j * CW, CW)],
            sem,
        )
        for j in range(CH)
    ]
    for cp in copies:
        cp.wait()
    pltpu.sync_copy(rows_v, out_hbm.at[pl.ds(wid * PER_W, PER_W)])


def _sc_gather(table, idx3):
    mesh = plsc.VectorSubcoreMesh(core_axis_name="c", subcore_axis_name="s")
    run = functools.partial(
        pl.kernel,
        out_type=jax.ShapeDtypeStruct((B * N * K, F), jnp.float32),
        mesh=mesh,
        scratch_types=[
            pltpu.VMEM((CH, CW), jnp.int32),
            pltpu.VMEM((PER_W, F), jnp.float32),
            pltpu.SemaphoreType.DMA,
        ],
        compiler_params=pltpu.CompilerParams(use_tc_tiling_on_sc=False),
    )(_gather_body)
    return run(table, idx3)


def _tc_consts():
    mm = jnp.arange(MB)
    sel = (mm[:, None] // K == jnp.arange(NB)[None, :]).astype(jnp.float32)
    p1 = (jnp.arange(OUTW)[None, :] // F
          == jnp.arange(CP)[:, None]).astype(jnp.float32)
    p2 = (jnp.arange(OUTW)[None, :] % F
          == jnp.arange(F)[:, None]).astype(jnp.float32)
    m1 = jnp.arange(M)
    blk = (m1[:, None] // K) == (m1[None, :] // K)
    w1 = (blk & (m1[:, None] != m1[None, :])).astype(jnp.float32)
    w2 = blk.astype(jnp.float32)
    amask = jnp.where(jnp.arange(CP) < C, 0.0, -1e30).reshape(1, CP)
    return sel, p1, p2, w1, w2, amask


def _tc_body(x_ref, g_ref, wwt_ref, bw_ref, wa1xt_ref, wa1yt_ref, ba1_ref,
             wa2t_ref, ba2_ref, sel_ref, p1_ref, p2_ref, w1_ref, w2_ref,
             amask_ref, out_ref, acc_ref):
    i = pl.program_id(0)

    @pl.when(i == 0)
    def _init():
        acc_ref[...] = jnp.zeros_like(acc_ref)

    x = x_ref[...]
    g = g_ref[...]
    wwt = wwt_ref[...]
    bw = bw_ref[...]

    def leaky(v):
        return jnp.maximum(v, 0.5 * v)

    def dot(a, b):
        return jnp.dot(a, b, preferred_element_type=jnp.float32)

    nt = (((1,), (1,)), ((), ()))

    def dot_nt(a, b):
        return lax.dot_general(a, b, nt, preferred_element_type=jnp.float32)

    wh = leaky(dot(x, wwt) + bw)
    whg = leaky(dot(g, wwt) + bw)

    px = dot(wh, wa1xt_ref[...]) + ba1_ref[...]
    py = dot(whg, wa1yt_ref[...])
    px_rep = dot(sel_ref[...], px)

    h = leaky(px_rep + py)
    att = leaky(dot(h, wa2t_ref[...]) + ba2_ref[...])

    att_m = att + amask_ref[...]
    mx = jnp.max(att_m, axis=1, keepdims=True)
    e = jnp.exp(att_m - mx)
    am = e / jnp.sum(e, axis=1, keepdims=True)

    ee = dot(am, p1_ref[...]) * dot(whg, p2_ref[...])
    out_ref[...] = jnp.sum(ee.reshape(NB, K, OUTW), axis=1)

    w1 = w1_ref[...]
    w2 = w2_ref[...]
    ones = jnp.ones((M, 1), dtype=jnp.float32)
    loss_t = 0.0
    dist_t = 0.0
    for b in range(RB):
        amb = am[b * M:(b + 1) * M]
        whb = whg[b * M:(b + 1) * M]
        prob = dot_nt(amb, amb)
        gram = dot_nt(whb, whb)
        sq = jnp.sum(whb * whb, axis=1, keepdims=True)
        sq_cols = dot_nt(ones, sq)
        dist = -2.0 * gram + sq + sq_cols
        sign = jnp.where(dist <= 0.2, 1.0, -1.0)
        lp = jnp.log(jnp.clip(prob, 0.0001, 1.0 - 0.0001)) * w1
        loss_t += jnp.sum(sign * lp)
        dist_t += jnp.sum(dist * w2)

    loss_sum = -loss_t
    wh_sum = jnp.sum(wh)

    acc_lane = lax.broadcasted_iota(jnp.int32, (1, 128), 1)
    vec = jnp.where(acc_lane == 0, loss_sum,
                    jnp.where(acc_lane == 1, dist_t,
                              jnp.where(acc_lane == 2, wh_sum, 0.0)))
    acc_ref[...] += vec


_CONST0 = lambda i: (0, 0)
_TC_GRID_SPEC = dict(
    grid=(GRID,),
    in_specs=[
        pl.BlockSpec((NB, F), lambda i: (i, 0)),
        pl.BlockSpec((MB, F), lambda i: (i, 0)),
        pl.BlockSpec((F, F), _CONST0),
        pl.BlockSpec((1, F), _CONST0),
        pl.BlockSpec((F, H), _CONST0),
        pl.BlockSpec((F, H), _CONST0),
        pl.BlockSpec((1, H), _CONST0),
        pl.BlockSpec((H, CP), _CONST0),
        pl.BlockSpec((1, CP), _CONST0),
        pl.BlockSpec((MB, NB), _CONST0),
        pl.BlockSpec((CP, OUTW), _CONST0),
        pl.BlockSpec((F, OUTW), _CONST0),
        pl.BlockSpec((M, M), _CONST0),
        pl.BlockSpec((M, M), _CONST0),
        pl.BlockSpec((1, CP), _CONST0),
    ],
    out_specs=[
        pl.BlockSpec((NB, OUTW), lambda i: (i, 0)),
        pl.BlockSpec((1, 128), _CONST0),
    ],
    out_shape=[
        jax.ShapeDtypeStruct((B * N, OUTW), jnp.float32),
        jax.ShapeDtypeStruct((1, 128), jnp.float32),
    ],
)


def _tc_compute(xpad, g, wwt, bw, wa1xt, wa1yt, ba1, wa2t, ba2):
    return pl.pallas_call(_tc_body, **_TC_GRID_SPEC)(
        xpad, g, wwt, bw, wa1xt, wa1yt, ba1, wa2t, ba2, *_tc_consts())


def kernel(fushed_features, input_data, Ww, bw, Wa1, ba1, Wa2, ba2, adj_idx):
    xpad = jnp.pad(input_data.reshape(B * N, SX), ((0, 0), (0, F - SX)))
    base = (jnp.arange(B, dtype=jnp.int32) * N)[:, None, None]
    idx3 = (adj_idx.astype(jnp.int32) + base).reshape(NW, CH, CW)

    g = _sc_gather(xpad, idx3)

    wwt = jnp.zeros((F, F), jnp.float32).at[:SX, :SO].set(Ww.T)
    bw_p = jnp.pad(bw, (0, F - SO)).reshape(1, F)
    wa1xt = jnp.zeros((F, H), jnp.float32).at[:SO].set(Wa1[:, :SO].T)
    wa1yt = jnp.zeros((F, H), jnp.float32).at[:SO].set(Wa1[:, SO:].T)
    ba1_p = ba1.reshape(1, H)
    wa2t = jnp.zeros((H, CP), jnp.float32).at[:, :C].set(Wa2.T)
    ba2_p = jnp.pad(ba2, (0, CP - C)).reshape(1, CP)

    out_main, acc = _tc_compute(xpad, g, wwt, bw_p, wa1xt, wa1yt, ba1_p,
                                wa2t, ba2_p)

    output_data = out_main.reshape(B * N, C, F)[:, :, :SO].reshape(B, N, C, SO)
    cluster_loss = acc[0, 0] / (B * N)
    dist_mean = acc[0, 1] / (B * N * K * K)
    wh_mean = acc[0, 2] / (B * N * SO)
    return output_data, cluster_loss, dist_mean, wh_mean


# speedup vs baseline: 5.9836x; 1.4788x over previous
import functools

import jax
import jax.numpy as jnp
from jax import lax
from jax.experimental import pallas as pl
from jax.experimental.pallas import tpu as pltpu
from jax.experimental.pallas import tpu_sc as plsc

B, N, K, C, SX, SO = 4, 400, 32, 6, 12, 12
F = 16
H = 48
CP = 8
R = 4
M = R * K
RB = 25
NB = RB * R
MB = NB * K
GRID = (B * N) // NB
OUTW = C * SO

NC, NS = 2, 16
NW = NC * NS
PER_W = (B * N * K) // NW
CH, CW = 16, 100


def _gather_body(table_hbm, idx_hbm, out_hbm, idx_v, rows_v, sem):
    wid = lax.axis_index("s") * NC + lax.axis_index("c")
    pltpu.sync_copy(idx_hbm.at[wid], idx_v)
    copies = [
        pltpu.async_copy(
            table_hbm.at[idx_v.at[j]],
            rows_v.at[pl.ds(j * CW, CW)],
            sem,
        )
        for j in range(CH)
    ]
    for cp in copies:
        cp.wait()
    pltpu.sync_copy(rows_v, out_hbm.at[pl.ds(wid * PER_W, PER_W)])


def _sc_gather(table, idx3):
    mesh = plsc.VectorSubcoreMesh(core_axis_name="c", subcore_axis_name="s")
    run = functools.partial(
        pl.kernel,
        out_type=jax.ShapeDtypeStruct((B * N * K, F), jnp.float32),
        mesh=mesh,
        scratch_types=[
            pltpu.VMEM((CH, CW), jnp.int32),
            pltpu.VMEM((PER_W, F), jnp.float32),
            pltpu.SemaphoreType.DMA,
        ],
        compiler_params=pltpu.CompilerParams(use_tc_tiling_on_sc=False),
    )(_gather_body)
    return run(table, idx3)


def _tc_consts():
    mm = jnp.arange(MB)
    sel_t = (jnp.arange(NB)[:, None] == mm[None, :] // K).astype(jnp.float32)
    m1 = jnp.arange(M)
    blk = (m1[:, None] // K) == (m1[None, :] // K)
    w1 = (blk & (m1[:, None] != m1[None, :])).astype(jnp.float32)
    amask = jnp.where(jnp.arange(CP) < C, 0.0, -1e30).reshape(CP, 1)
    return sel_t, w1, amask


def _tc_body(xt_ref, gt_ref, ww_ref, bw_ref, wa1x_ref, wa1y_ref, ba1_ref,
             wa2_ref, ba2_ref, selt_ref, w1_ref, amask_ref, out_ref, acc_ref):
    i = pl.program_id(0)

    @pl.when(i == 0)
    def _init():
        acc_ref[...] = jnp.zeros_like(acc_ref)

    def leaky(v):
        return jnp.maximum(v, 0.5 * v)

    def dot(a, b):
        return jnp.dot(a, b, preferred_element_type=jnp.float32)

    def dot_nt(a, b):
        return lax.dot_general(a, b, (((1,), (1,)), ((), ())),
                               preferred_element_type=jnp.float32)

    def dot_tn(a, b):
        return lax.dot_general(a, b, (((0,), (0,)), ((), ())),
                               preferred_element_type=jnp.float32)

    xt = xt_ref[0]
    gt = gt_ref[...]
    ww = ww_ref[...]
    bw = bw_ref[...]
    sel_t = selt_ref[...]

    wht = leaky(dot(ww, xt) + bw)
    whgt = leaky(dot(ww, gt) + bw)

    pxt = dot(wa1x_ref[...], wht) + ba1_ref[...]
    pxrep = dot(pxt, sel_t)
    pyt = dot(wa1y_ref[...], whgt)
    ht = leaky(pxrep + pyt)
    att = leaky(dot(wa2_ref[...], ht) + ba2_ref[...])

    att_m = att + amask_ref[...]
    mx = jnp.max(att_m, axis=0, keepdims=True)
    e = jnp.exp(att_m - mx)
    amt = e / jnp.sum(e, axis=0, keepdims=True)

    outs = [dot_nt(whgt[:SO] * amt[c:c + 1], sel_t) for c in range(C)]
    out_ref[0] = jnp.concatenate(outs, axis=0)

    ns = dot_nt(whgt, sel_t)
    sq = jnp.sum(whgt * whgt, axis=0, keepdims=True)
    dist_t = -2.0 * jnp.sum(ns * ns) + 2.0 * K * jnp.sum(sq)

    w1 = w1_ref[...]
    one11 = jnp.ones((1, 1), dtype=jnp.float32)
    loss_t = 0.0
    for b in range(RB):
        amb = amt[:, b * M:(b + 1) * M]
        whb = whgt[:, b * M:(b + 1) * M]
        sqb = sq[:, b * M:(b + 1) * M]
        prob = dot_tn(amb, amb)
        gram = dot_tn(whb, whb)
        sqcol = dot_tn(sqb, one11)
        dist = -2.0 * gram + sqcol + sqb
        sign = jnp.where(dist <= 0.2, 1.0, -1.0)
        lp = jnp.log(jnp.clip(prob, 0.0001, 1.0 - 0.0001)) * w1
        loss_t += jnp.sum(sign * lp)

    loss_sum = -loss_t
    wh_sum = jnp.sum(wht)

    acc_lane = lax.broadcasted_iota(jnp.int32, (1, 128), 1)
    vec = jnp.where(acc_lane == 0, loss_sum,
                    jnp.where(acc_lane == 1, dist_t,
                              jnp.where(acc_lane == 2, wh_sum, 0.0)))
    acc_ref[...] += vec


_CONST0 = lambda i: (0, 0)
_TC_GRID_SPEC = dict(
    grid=(GRID,),
    in_specs=[
        pl.BlockSpec((1, F, NB), lambda i: (i, 0, 0)),
        pl.BlockSpec((F, MB), lambda i: (0, i)),
        pl.BlockSpec((F, F), _CONST0),
        pl.BlockSpec((F, 1), _CONST0),
        pl.BlockSpec((H, F), _CONST0),
        pl.BlockSpec((H, F), _CONST0),
        pl.BlockSpec((H, 1), _CONST0),
        pl.BlockSpec((CP, H), _CONST0),
        pl.BlockSpec((CP, 1), _CONST0),
        pl.BlockSpec((NB, MB), _CONST0),
        pl.BlockSpec((M, M), _CONST0),
        pl.BlockSpec((CP, 1), _CONST0),
    ],
    out_specs=[
        pl.BlockSpec((1, OUTW, NB), lambda i: (i, 0, 0)),
        pl.BlockSpec((1, 128), _CONST0),
    ],
    out_shape=[
        jax.ShapeDtypeStruct((GRID, OUTW, NB), jnp.float32),
        jax.ShapeDtypeStruct((1, 128), jnp.float32),
    ],
)


def _tc_compute(xt, gt, ww, bw, wa1x, wa1y, ba1, wa2, ba2):
    return pl.pallas_call(_tc_body, **_TC_GRID_SPEC)(
        xt, gt, ww, bw, wa1x, wa1y, ba1, wa2, ba2, *_tc_consts())


def kernel(fushed_features, input_data, Ww, bw, Wa1, ba1, Wa2, ba2, adj_idx):
    xpad = jnp.pad(input_data.reshape(B * N, SX), ((0, 0), (0, F - SX)))
    base = (jnp.arange(B, dtype=jnp.int32) * N)[:, None, None]
    idx3 = (adj_idx.astype(jnp.int32) + base).reshape(NW, CH, CW)

    g = _sc_gather(xpad, idx3)

    xt3 = xpad.reshape(GRID, NB, F).swapaxes(1, 2)
    gt = g.T
    ww = jnp.pad(Ww, ((0, F - SO), (0, F - SX)))
    bw_p = jnp.pad(bw, (0, F - SO)).reshape(F, 1)
    wa1x = jnp.pad(Wa1[:, :SO], ((0, 0), (0, F - SO)))
    wa1y = jnp.pad(Wa1[:, SO:], ((0, 0), (0, F - SO)))
    ba1_p = ba1.reshape(H, 1)
    wa2 = jnp.pad(Wa2, ((0, CP - C), (0, 0)))
    ba2_p = jnp.pad(ba2, (0, CP - C)).reshape(CP, 1)

    out3, acc = _tc_compute(xt3, gt, ww, bw_p, wa1x, wa1y, ba1_p, wa2,
                            ba2_p)

    output_data = out3.swapaxes(1, 2).reshape(B, N, C, SO)
    cluster_loss = acc[0, 0] / (B * N)
    dist_mean = acc[0, 1] / (B * N * K * K)
    wh_mean = acc[0, 2] / (B * N * SO)
    return output_data, cluster_loss, dist_mean, wh_mean


# speedup vs baseline: 6.4861x; 1.0840x over previous
import functools

import jax
import jax.numpy as jnp
from jax import lax
from jax.experimental import pallas as pl
from jax.experimental.pallas import tpu as pltpu
from jax.experimental.pallas import tpu_sc as plsc

B, N, K, C, SX, SO = 4, 400, 32, 6, 12, 12
F = 16
H = 48
R = 4
M = R * K
RB = 25
NB = RB * R
MB = NB * K
GRID = (B * N) // NB
OUTW = C * SO

NC, NS = 2, 16
NW = NC * NS
PER_W = (B * N * K) // NW
CH, CW = 16, 100


def _gather_body(table_hbm, idx_hbm, out_hbm, idx_v, rows_v, sem):
    wid = lax.axis_index("s") * NC + lax.axis_index("c")
    pltpu.sync_copy(idx_hbm.at[wid], idx_v)
    copies = [
        pltpu.async_copy(
            table_hbm.at[idx_v.at[j]],
            rows_v.at[pl.ds(j * CW, CW)],
            sem,
        )
        for j in range(CH)
    ]
    for cp in copies:
        cp.wait()
    pltpu.sync_copy(rows_v, out_hbm.at[pl.ds(wid * PER_W, PER_W)])


def _sc_gather(table, idx3):
    mesh = plsc.VectorSubcoreMesh(core_axis_name="c", subcore_axis_name="s")
    run = functools.partial(
        pl.kernel,
        out_type=jax.ShapeDtypeStruct((B * N * K, F), jnp.float32),
        mesh=mesh,
        scratch_types=[
            pltpu.VMEM((CH, CW), jnp.int32),
            pltpu.VMEM((PER_W, F), jnp.float32),
            pltpu.SemaphoreType.DMA,
        ],
        compiler_params=pltpu.CompilerParams(use_tc_tiling_on_sc=False),
    )(_gather_body)
    return run(table, idx3)


def _pack_weights(Ww, bw, Wa1, ba1, Wa2, ba2):
    wp = jnp.zeros((128, 64), jnp.float32)
    wp = wp.at[0:SO, 0:SX].set(Ww)
    wp = wp.at[16:16 + H, 0:SO].set(Wa1[:, :SO])
    wp = wp.at[64:64 + H, 0:SO].set(Wa1[:, SO:])
    wp = wp.at[112:112 + C, 0:H].set(Wa2)
    wp = wp.at[0:SO, 63].set(bw)
    wp = wp.at[16:16 + H, 63].set(ba1)
    wp = wp.at[112:112 + C, 63].set(ba2)
    return wp


def _tc_consts():
    mm = jnp.arange(MB)
    sel_t = (jnp.arange(NB)[:, None] == mm[None, :] // K).astype(jnp.float32)
    m1 = jnp.arange(M)
    blk = (m1[:, None] // K) == (m1[None, :] // K)
    w1 = (blk & (m1[:, None] != m1[None, :])).astype(jnp.float32)
    return sel_t, w1


def _tc_body(x_ref, g_ref, wp_ref, selt_ref, w1_ref, out_ref, acc_ref):
    i = pl.program_id(0)

    @pl.when(i == 0)
    def _init():
        acc_ref[...] = jnp.zeros_like(acc_ref)

    def leaky(v):
        return jnp.maximum(v, 0.5 * v)

    def dot(a, b):
        return jnp.dot(a, b, preferred_element_type=jnp.float32)

    def dot_nt(a, b):
        return lax.dot_general(a, b, (((1,), (1,)), ((), ())),
                               preferred_element_type=jnp.float32)

    def dot_tn(a, b):
        return lax.dot_general(a, b, (((0,), (0,)), ((), ())),
                               preferred_element_type=jnp.float32)

    x = x_ref[0]
    g = g_ref[0]
    sel_t = selt_ref[...]

    ww = wp_ref[0:SO, 0:F]
    wa1x = wp_ref[16:16 + H, 0:SO]
    wa1y = wp_ref[64:64 + H, 0:SO]
    wa2 = wp_ref[112:112 + C, 0:H]
    bw = wp_ref[0:SO, 63:64]
    ba1 = wp_ref[16:16 + H, 63:64]
    ba2 = wp_ref[112:112 + C, 63:64]

    wht = leaky(dot_nt(ww, x) + bw)
    whgt = leaky(dot_nt(ww, g) + bw)

    pxt = dot(wa1x, wht) + ba1
    pxrep = dot(pxt, sel_t)
    pyt = dot(wa1y, whgt)
    ht = leaky(pxrep + pyt)
    att = leaky(dot(wa2, ht) + ba2)

    mx = jnp.max(att, axis=0, keepdims=True)
    e = jnp.exp(att - mx)
    amt = e / jnp.sum(e, axis=0, keepdims=True)

    outs = [dot_nt(sel_t, whgt * amt[c:c + 1]) for c in range(C)]
    out_ref[0] = jnp.concatenate(outs, axis=1)

    ns = dot_nt(sel_t, whgt)
    sq = jnp.sum(whgt * whgt, axis=0, keepdims=True)
    dist_t = -2.0 * jnp.sum(ns * ns) + 2.0 * K * jnp.sum(sq)

    w1 = w1_ref[...]
    one11 = jnp.ones((1, 1), dtype=jnp.float32)
    loss_t = 0.0
    for b in range(RB):
        amb = amt[:, b * M:(b + 1) * M]
        whb = whgt[:, b * M:(b + 1) * M]
        sqb = sq[:, b * M:(b + 1) * M]
        prob = dot_tn(amb, amb)
        gram = dot_tn(whb, whb)
        sqcol = dot_tn(sqb, one11)
        dist = -2.0 * gram + sqcol + sqb
        sign = jnp.where(dist <= 0.2, 1.0, -1.0)
        lp = jnp.log(jnp.clip(prob, 0.0001, 1.0 - 0.0001)) * w1
        loss_t += jnp.sum(sign * lp)

    loss_sum = -loss_t
    wh_sum = jnp.sum(wht)

    acc_lane = lax.broadcasted_iota(jnp.int32, (1, 128), 1)
    vec = jnp.where(acc_lane == 0, loss_sum,
                    jnp.where(acc_lane == 1, dist_t,
                              jnp.where(acc_lane == 2, wh_sum, 0.0)))
    acc_ref[...] += vec


_CONST0 = lambda i: (0, 0)
_TC_GRID_SPEC = dict(
    grid=(GRID,),
    in_specs=[
        pl.BlockSpec((1, NB, F), lambda i: (i, 0, 0)),
        pl.BlockSpec((1, MB, F), lambda i: (i, 0, 0)),
        pl.BlockSpec((128, 64), _CONST0),
        pl.BlockSpec((NB, MB), _CONST0),
        pl.BlockSpec((M, M), _CONST0),
    ],
    out_specs=[
        pl.BlockSpec((1, NB, OUTW), lambda i: (i, 0, 0)),
        pl.BlockSpec((1, 128), _CONST0),
    ],
    out_shape=[
        jax.ShapeDtypeStruct((GRID, NB, OUTW), jnp.float32),
        jax.ShapeDtypeStruct((1, 128), jnp.float32),
    ],
)


def _tc_compute(x3, g3, wp):
    return pl.pallas_call(_tc_body, **_TC_GRID_SPEC)(
        x3, g3, wp, *_tc_consts())


def kernel(fushed_features, input_data, Ww, bw, Wa1, ba1, Wa2, ba2, adj_idx):
    xpad = jnp.pad(input_data.reshape(B * N, SX), ((0, 0), (0, F - SX)))
    base = (jnp.arange(B, dtype=jnp.int32) * N)[:, None, None]
    idx3 = (adj_idx.astype(jnp.int32) + base).reshape(NW, CH, CW)

    g = _sc_gather(xpad, idx3)

    x3 = xpad.reshape(GRID, NB, F)
    g3 = g.reshape(GRID, MB, F)
    wp = _pack_weights(Ww, bw, Wa1, ba1, Wa2, ba2)

    out3, acc = _tc_compute(x3, g3, wp)

    output_data = out3.reshape(B, N, C, SO)
    cluster_loss = acc[0, 0] / (B * N)
    dist_mean = acc[0, 1] / (B * N * K * K)
    wh_mean = acc[0, 2] / (B * N * SO)
    return output_data, cluster_loss, dist_mean, wh_mean
